# baseline (speedup 1.0000x reference)
import functools

import jax
import jax.numpy as jnp
from jax import lax
from jax.experimental import pallas as pl
from jax.experimental.pallas import tpu as pltpu
from jax.experimental.pallas import tpu_sc as plsc

NC = 2
NS = 16
LANE = 16


def _mm_aug_body(x_ref, w_ref, o_ref):
    m = jnp.dot(x_ref[...], w_ref[...], preferred_element_type=jnp.float32)
    ones = jnp.ones((m.shape[0], LANE), jnp.float32)
    o_ref[...] = jnp.concatenate([m, ones], axis=1)


def _mm_aug(x, w, rb):
    n, d = x.shape
    hc = w.shape[1] // 2
    nb = n // rb
    return pl.pallas_call(
        _mm_aug_body,
        grid=(2, nb),
        in_specs=[
            pl.BlockSpec((rb, d), lambda c, r: (r, 0)),
            pl.BlockSpec((d, hc), lambda c, r: (0, c)),
        ],
        out_specs=pl.BlockSpec((rb, hc + LANE), lambda c, r: (c * nb + r, 0)),
        out_shape=jax.ShapeDtypeStruct((2 * n, hc + LANE), jnp.float32),
    )(x, w)


def _fold_body(wf_ref, wl_ref, wc_ref, bf_ref, bl_ref, bc_ref,
               of_ref, ol_ref, oc_ref):
    wc = wc_ref[...]
    of_ref[...] = jnp.dot(wf_ref[...], wc, preferred_element_type=jnp.float32)
    ol_ref[...] = jnp.dot(wl_ref[...], wc, preferred_element_type=jnp.float32)
    bsum = 0.5 * (bf_ref[...] + bl_ref[...])
    oc_ref[...] = jnp.dot(bsum, wc, preferred_element_type=jnp.float32) + bc_ref[...]


def _fold(w1f, w1l, wc, b1f, b1l, bc):
    hd, od = wc.shape
    return pl.pallas_call(
        _fold_body,
        out_shape=(
            jax.ShapeDtypeStruct((hd, od), jnp.float32),
            jax.ShapeDtypeStruct((hd, od), jnp.float32),
            jax.ShapeDtypeStruct((1, od), jnp.float32),
        ),
    )(w1f, w1l, wc, b1f.reshape(1, -1), b1l.reshape(1, -1), bc.reshape(1, -1))


def _relu2mm(aggf, aggl, b0f, b0l, wp, rb):
    _, n, r = aggf.shape
    hc = r - LANE
    od = wp.shape[1]
    nb = n // rb

    def body(af_ref, al_ref, bf_ref, bl_ref, w_ref, o_ref):
        acc = jnp.zeros((rb, od), jnp.float32)
        for c in range(2):
            af = af_ref[c]
            al = al_ref[c]
            invf = 1.0 / jnp.maximum(af[:, hc], 1.0)
            invl = 1.0 / jnp.maximum(al[:, hc], 1.0)
            hu = 0.5 * (af[:, :hc] * invf[:, None] + bf_ref[0, pl.ds(c * hc, hc)][None, :]
                        + al[:, :hc] * invl[:, None] + bl_ref[0, pl.ds(c * hc, hc)][None, :])
            hu = jnp.maximum(hu, 0.0)
            acc = acc + jnp.dot(hu, w_ref[pl.ds(c * hc, hc), :],
                                preferred_element_type=jnp.float32)
        o_ref[...] = acc

    return pl.pallas_call(
        body,
        grid=(nb,),
        in_specs=[
            pl.BlockSpec((2, rb, r), lambda i: (0, i, 0)),
            pl.BlockSpec((2, rb, r), lambda i: (0, i, 0)),
            pl.BlockSpec((1, 2 * hc), lambda i: (0, 0)),
            pl.BlockSpec((1, 2 * hc), lambda i: (0, 0)),
            pl.BlockSpec((2 * hc, od), lambda i: (0, 0)),
        ],
        out_specs=pl.BlockSpec((rb, od), lambda i: (i, 0)),
        out_shape=jax.ShapeDtypeStruct((n, od), jnp.float32),
    )(aggf, aggl, b0f.reshape(1, -1), b0l.reshape(1, -1), wp)


def _relu1mm(aggr, b0r, wp, rb):
    _, n, r = aggr.shape
    hc = r - LANE
    od = wp.shape[1]
    nb = n // rb

    def body(ar_ref, br_ref, w_ref, o_ref):
        acc = jnp.zeros((rb, od), jnp.float32)
        for c in range(2):
            ar = ar_ref[c]
            inv = 1.0 / jnp.maximum(ar[:, hc], 1.0)
            hi = ar[:, :hc] * inv[:, None] + br_ref[0, pl.ds(c * hc, hc)][None, :]
            hi = jnp.maximum(hi, 0.0)
            acc = acc + jnp.dot(hi, w_ref[pl.ds(c * hc, hc), :],
                                preferred_element_type=jnp.float32)
        o_ref[...] = acc

    return pl.pallas_call(
        body,
        grid=(nb,),
        in_specs=[
            pl.BlockSpec((2, rb, r), lambda i: (0, i, 0)),
            pl.BlockSpec((1, 2 * hc), lambda i: (0, 0)),
            pl.BlockSpec((2 * hc, od), lambda i: (0, 0)),
        ],
        out_specs=pl.BlockSpec((rb, od), lambda i: (i, 0)),
        out_shape=jax.ShapeDtypeStruct((n, od), jnp.float32),
    )(aggr, b0r.reshape(1, -1), wp)


def _final(a2f, a2l, aggf, aggl, crow, rb):
    _, n, od = a2f.shape
    r1 = aggf.shape[2]
    hc = r1 - LANE
    nb = n // rb

    def body(f_ref, l_ref, df_ref, dl_ref, c_ref, o_ref):
        invf = 1.0 / jnp.maximum(df_ref[0, :, hc], 1.0)
        invl = 1.0 / jnp.maximum(dl_ref[0, :, hc], 1.0)
        s = 0.5 * ((f_ref[0] + f_ref[1]) * invf[:, None]
                   + (l_ref[0] + l_ref[1]) * invl[:, None])
        o_ref[...] = s + c_ref[0][None, :]

    return pl.pallas_call(
        body,
        grid=(nb,),
        in_specs=[
            pl.BlockSpec((2, rb, od), lambda i: (0, i, 0)),
            pl.BlockSpec((2, rb, od), lambda i: (0, i, 0)),
            pl.BlockSpec((1, rb, r1), lambda i: (0, i, 0)),
            pl.BlockSpec((1, rb, r1), lambda i: (0, i, 0)),
            pl.BlockSpec((1, od), lambda i: (0, 0)),
        ],
        out_specs=pl.BlockSpec((rb, od), lambda i: (i, 0)),
        out_shape=jax.ShapeDtypeStruct((n, od), jnp.float32),
    )(a2f, a2l, aggf, aggl, crow)


NBUF = 2
SEG1 = 8


def _sc_segsum_multi(ms, srcbs, dstbs, n_nodes, r, nbuf, seg):
    ne = len(ms)
    bps = srcbs[0].shape[1] // NS
    nseg = bps // seg
    assert bps % seg == 0 and seg % nbuf == 0 and seg >= 2 * nbuf
    assert nseg == 1 or nseg % 2 == 0
    npad = n_nodes + 8
    rows_zero = -(-npad // NS)
    rows_out = n_nodes // NS
    nplane = 1 if nbuf == 1 else 2
    mesh = plsc.VectorSubcoreMesh(core_axis_name="c", subcore_axis_name="s")

    @functools.partial(
        pl.kernel,
        mesh=mesh,
        compiler_params=pltpu.CompilerParams(use_tc_tiling_on_sc=False),
        out_type=tuple(jax.ShapeDtypeStruct((2, n_nodes, r), jnp.float32)
                       for _ in range(ne)),
        scratch_types=[
            pltpu.VMEM_SHARED((npad, r), jnp.float32),
            pltpu.VMEM((nplane, seg, 128), jnp.int32),
            pltpu.VMEM((nplane, seg, 128), jnp.int32),
        ]
        + [pltpu.VMEM((128, r), jnp.float32) for _ in range(nbuf)]
        + [pltpu.SemaphoreType.DMA for _ in range(2 * nbuf)]
        + [pltpu.SemaphoreType.DMA, pltpu.SemaphoreType.DMA],
    )
    def k(*refs):
        ins = refs[:3 * ne]
        outs = refs[3 * ne:4 * ne]
        acc_sh, idxs, idxd = refs[4 * ne:4 * ne + 3]
        rest = refs[4 * ne + 3:]
        gbufs = rest[:nbuf]
        gsems = rest[nbuf:2 * nbuf]
        ssems = rest[2 * nbuf:3 * nbuf]
        isems = rest[3 * nbuf:]
        c = lax.axis_index("c")
        s = lax.axis_index("s")
        zbase = jnp.minimum(s * rows_zero, npad - rows_zero)
        b0 = s * bps

        def maybe_when(cond, fn):
            if isinstance(cond, bool):
                if cond:
                    fn()
            else:
                pl.when(cond)(fn)

        def one_etype(m_hbm, srcb_hbm, dstb_hbm, out_hbm):
            def i_start(g, p):
                pltpu.async_copy(srcb_hbm.at[c].at[pl.ds(b0 + g * seg, seg)],
                                 idxs.at[p], isems[0])
                pltpu.async_copy(dstb_hbm.at[c].at[pl.ds(b0 + g * seg, seg)],
                                 idxd.at[p], isems[1])

            def i_wait(g, p):
                pltpu.make_async_copy(srcb_hbm.at[c].at[pl.ds(b0 + g * seg, seg)],
                                      idxs.at[p], isems[0]).wait()
                pltpu.make_async_copy(dstb_hbm.at[c].at[pl.ds(b0 + g * seg, seg)],
                                      idxd.at[p], isems[1]).wait()

            i_start(0, 0)

            @pl.loop(0, 128)
            def _(i):
                @pl.loop(0, r, step=LANE)
                def _(j):
                    gbufs[0][i, pl.ds(j, LANE)] = jnp.zeros((LANE,),
                                                            jnp.float32)

            off = 0
            left = rows_zero
            while left > 0:
                nn = min(128, left)
                pltpu.sync_copy(gbufs[0].at[pl.ds(0, nn)],
                                acc_sh.at[pl.ds(zbase + off, nn)])
                off += nn
                left -= nn
            plsc.subcore_barrier()
            i_wait(0, 0)

            if nbuf == 1:
                @pl.loop(0, bps)
                def _(j):
                    pltpu.sync_copy(m_hbm.at[idxs.at[0].at[j]], gbufs[0])
                    pltpu.sync_copy(gbufs[0], acc_sh.at[idxd.at[0].at[j]],
                                    add=True)
            else:
                def g_start(p, jj, b):
                    pltpu.async_copy(m_hbm.at[idxs.at[p].at[jj]], gbufs[b],
                                     gsems[b])

                def g_wait(p, jj, b):
                    pltpu.make_async_copy(m_hbm.at[idxs.at[p].at[jj]],
                                          gbufs[b], gsems[b]).wait()

                def s_start(p, jj, b):
                    pltpu.async_copy(gbufs[b], acc_sh.at[idxd.at[p].at[jj]],
                                     ssems[b], add=True)

                def s_wait(p, jj, b):
                    pltpu.make_async_copy(gbufs[b],
                                          acc_sh.at[idxd.at[p].at[jj]],
                                          ssems[b]).wait()

                def seg_body(g, p, pn, first):
                    for jj in range(seg):
                        b = jj % nbuf
                        g_wait(p, jj, b)
                        s_start(p, jj, b)
                        bn = (jj + 1) % nbuf
                        if first and jj + 1 < nbuf:
                            g_start(p, jj + 1, bn)
                        elif jj + 1 < seg:
                            jprev = jj + 1 - nbuf
                            if first or jprev >= 0:
                                s_wait(p, jprev, bn)
                            else:
                                s_wait(pn, jprev + seg, bn)
                            g_start(p, jj + 1, bn)
                        else:
                            def _refill():
                                i_wait(g + 1, pn)
                                s_wait(p, seg - nbuf, bn)
                                g_start(pn, 0, bn)
                            maybe_when(g < nseg - 1, _refill)
                        if jj == nbuf - 1 and nseg > 1:
                            if first:
                                i_start(1, pn)
                            else:
                                maybe_when(g < nseg - 1,
                                           lambda: i_start(g + 1, pn))

                g_start(0, 0, 0)
                seg_body(0, 0, 1, True)
                if nseg > 2:
                    @pl.loop(1, nseg - 1, step=2)
                    def _(g):
                        seg_body(g, 1, 0, False)
                        seg_body(g + 1, 0, 1, False)
                if nseg > 1:
                    seg_body(nseg - 1, (nseg - 1) % 2, nseg % 2, False)

                for b2 in range(nbuf):
                    jj = seg - nbuf + b2
                    s_wait((nseg - 1) % 2, jj, jj % nbuf)

            plsc.subcore_barrier()
            obase = s * rows_out
            pltpu.sync_copy(acc_sh.at[pl.ds(obase, rows_out)],
                            out_hbm.at[c].at[pl.ds(obase, rows_out)])

        for t in range(ne):
            one_etype(ins[3 * t], ins[3 * t + 1], ins[3 * t + 2], outs[t])
            if t < ne - 1:
                plsc.subcore_barrier()

    return k(*[x for trip in zip(ms, srcbs, dstbs) for x in trip])


def _prep_idx(edge, n_nodes, split_edges, blk_mult):
    src, dst = edge[0], edge[1]
    e = src.shape[0]
    tot_workers = NS * (NC if split_edges else 1)
    per = -(-e // (tot_workers * 128 * blk_mult)) * 128 * blk_mult
    pad = per * tot_workers - e
    src_p = jnp.concatenate([src, jnp.zeros((pad,), jnp.int32)])
    dst_p = jnp.concatenate([dst, jnp.full((pad,), n_nodes, jnp.int32)])
    if split_edges:
        srcb = src_p.reshape(2, -1, 128)
        dstb = dst_p.reshape(2, -1, 128)
    else:
        srcb = jnp.stack([src_p, src_p + n_nodes]).reshape(2, -1, 128)
        dstb = jnp.stack([dst_p, dst_p]).reshape(2, -1, 128)
    return srcb, dstb


def kernel(x_user, x_item, edge_follows, edge_likes, edge_rev_likes,
           W0_follows, b0_follows, W0_likes, b0_likes, W0_rev_likes, b0_rev_likes,
           W1_follows, b1_follows, W1_likes, b1_likes, W1_rev_likes, b1_rev_likes,
           W_cls, b_cls):
    n = x_user.shape[0]
    h = W0_follows.shape[1]
    rb = 1000
    r1 = h // 2 + LANE

    mf = _mm_aug(x_user, W0_follows, rb)
    ml = _mm_aug(x_item, W0_likes, rb)
    mr = _mm_aug(x_user, W0_rev_likes, rb)

    sf1, df1 = _prep_idx(edge_follows, n, False, 1)
    sl1, dl1 = _prep_idx(edge_likes, n, False, 1)
    sr1, dr1 = _prep_idx(edge_rev_likes, n, False, 1)
    bps1 = sf1.shape[1] // NS
    aggf, aggl, aggr = _sc_segsum_multi(
        [mf, ml, mr], [sf1, sl1, sr1], [df1, dl1, dr1], n, r1, 1, bps1)

    wfp, wlp, crow = _fold(W1_follows, W1_likes, W_cls, b1_follows, b1_likes, b_cls)
    m2f = _relu2mm(aggf, aggl, b0_follows, b0_likes, wfp, rb)
    m2l = _relu1mm(aggr, b0_rev_likes, wlp, rb)

    sf2, df2 = _prep_idx(edge_follows, n, True, 4)
    sl2, dl2 = _prep_idx(edge_likes, n, True, 4)
    bps2 = sf2.shape[1] // NS
    a2f, a2l = _sc_segsum_multi(
        [m2f, m2l], [sf2, sl2], [df2, dl2], n, W_cls.shape[1], 4, bps2)

    return _final(a2f, a2l, aggf, aggl, crow, rb)


# speedup vs baseline: 1.2091x; 1.2091x over previous
import functools

import jax
import jax.numpy as jnp
from jax import lax
from jax.experimental import pallas as pl
from jax.experimental.pallas import tpu as pltpu
from jax.experimental.pallas import tpu_sc as plsc

NC = 2
NS = 16
LANE = 16


def _mm_aug_body(x_ref, w_ref, o_ref):
    m = jnp.dot(x_ref[...], w_ref[...], preferred_element_type=jnp.float32)
    ones = jnp.ones((m.shape[0], LANE), jnp.float32)
    o_ref[...] = jnp.concatenate([m, ones], axis=1)


def _mm_aug(x, w, rb):
    n, d = x.shape
    hc = w.shape[1] // 2
    nb = n // rb
    return pl.pallas_call(
        _mm_aug_body,
        grid=(2, nb),
        in_specs=[
            pl.BlockSpec((rb, d), lambda c, r: (r, 0)),
            pl.BlockSpec((d, hc), lambda c, r: (0, c)),
        ],
        out_specs=pl.BlockSpec((rb, hc + LANE), lambda c, r: (c * nb + r, 0)),
        out_shape=jax.ShapeDtypeStruct((2 * n, hc + LANE), jnp.float32),
    )(x, w)


def _fold_body(wf_ref, wl_ref, wc_ref, bf_ref, bl_ref, bc_ref,
               of_ref, ol_ref, oc_ref):
    wc = wc_ref[...]
    of_ref[...] = jnp.dot(wf_ref[...], wc, preferred_element_type=jnp.float32)
    ol_ref[...] = jnp.dot(wl_ref[...], wc, preferred_element_type=jnp.float32)
    bsum = 0.5 * (bf_ref[...] + bl_ref[...])
    oc_ref[...] = jnp.dot(bsum, wc, preferred_element_type=jnp.float32) + bc_ref[...]


def _fold(w1f, w1l, wc, b1f, b1l, bc):
    hd, od = wc.shape
    return pl.pallas_call(
        _fold_body,
        out_shape=(
            jax.ShapeDtypeStruct((hd, od), jnp.float32),
            jax.ShapeDtypeStruct((hd, od), jnp.float32),
            jax.ShapeDtypeStruct((1, od), jnp.float32),
        ),
    )(w1f, w1l, wc, b1f.reshape(1, -1), b1l.reshape(1, -1), bc.reshape(1, -1))


def _relu2mm(aggf, aggl, b0f, b0l, wp, rb):
    _, n, r = aggf.shape
    hc = r - LANE
    od = wp.shape[1]
    nb = n // rb

    def body(af_ref, al_ref, bf_ref, bl_ref, w_ref, o_ref):
        acc = jnp.zeros((rb, od), jnp.float32)
        for c in range(2):
            af = af_ref[c]
            al = al_ref[c]
            invf = 1.0 / jnp.maximum(af[:, hc], 1.0)
            invl = 1.0 / jnp.maximum(al[:, hc], 1.0)
            hu = 0.5 * (af[:, :hc] * invf[:, None] + bf_ref[0, pl.ds(c * hc, hc)][None, :]
                        + al[:, :hc] * invl[:, None] + bl_ref[0, pl.ds(c * hc, hc)][None, :])
            hu = jnp.maximum(hu, 0.0)
            acc = acc + jnp.dot(hu, w_ref[pl.ds(c * hc, hc), :],
                                preferred_element_type=jnp.float32)
        o_ref[...] = acc

    return pl.pallas_call(
        body,
        grid=(nb,),
        in_specs=[
            pl.BlockSpec((2, rb, r), lambda i: (0, i, 0)),
            pl.BlockSpec((2, rb, r), lambda i: (0, i, 0)),
            pl.BlockSpec((1, 2 * hc), lambda i: (0, 0)),
            pl.BlockSpec((1, 2 * hc), lambda i: (0, 0)),
            pl.BlockSpec((2 * hc, od), lambda i: (0, 0)),
        ],
        out_specs=pl.BlockSpec((rb, od), lambda i: (i, 0)),
        out_shape=jax.ShapeDtypeStruct((n, od), jnp.float32),
    )(aggf, aggl, b0f.reshape(1, -1), b0l.reshape(1, -1), wp)


def _relu1mm(aggr, b0r, wp, rb):
    _, n, r = aggr.shape
    hc = r - LANE
    od = wp.shape[1]
    nb = n // rb

    def body(ar_ref, br_ref, w_ref, o_ref):
        acc = jnp.zeros((rb, od), jnp.float32)
        for c in range(2):
            ar = ar_ref[c]
            inv = 1.0 / jnp.maximum(ar[:, hc], 1.0)
            hi = ar[:, :hc] * inv[:, None] + br_ref[0, pl.ds(c * hc, hc)][None, :]
            hi = jnp.maximum(hi, 0.0)
            acc = acc + jnp.dot(hi, w_ref[pl.ds(c * hc, hc), :],
                                preferred_element_type=jnp.float32)
        o_ref[...] = acc

    return pl.pallas_call(
        body,
        grid=(nb,),
        in_specs=[
            pl.BlockSpec((2, rb, r), lambda i: (0, i, 0)),
            pl.BlockSpec((1, 2 * hc), lambda i: (0, 0)),
            pl.BlockSpec((2 * hc, od), lambda i: (0, 0)),
        ],
        out_specs=pl.BlockSpec((rb, od), lambda i: (i, 0)),
        out_shape=jax.ShapeDtypeStruct((n, od), jnp.float32),
    )(aggr, b0r.reshape(1, -1), wp)


def _final(a2f, a2l, aggf, aggl, crow, rb):
    _, n, od = a2f.shape
    r1 = aggf.shape[2]
    hc = r1 - LANE
    nb = n // rb

    def body(f_ref, l_ref, df_ref, dl_ref, c_ref, o_ref):
        invf = 1.0 / jnp.maximum(df_ref[0, :, hc], 1.0)
        invl = 1.0 / jnp.maximum(dl_ref[0, :, hc], 1.0)
        s = 0.5 * ((f_ref[0] + f_ref[1]) * invf[:, None]
                   + (l_ref[0] + l_ref[1]) * invl[:, None])
        o_ref[...] = s + c_ref[0][None, :]

    return pl.pallas_call(
        body,
        grid=(nb,),
        in_specs=[
            pl.BlockSpec((2, rb, od), lambda i: (0, i, 0)),
            pl.BlockSpec((2, rb, od), lambda i: (0, i, 0)),
            pl.BlockSpec((1, rb, r1), lambda i: (0, i, 0)),
            pl.BlockSpec((1, rb, r1), lambda i: (0, i, 0)),
            pl.BlockSpec((1, od), lambda i: (0, 0)),
        ],
        out_specs=pl.BlockSpec((rb, od), lambda i: (i, 0)),
        out_shape=jax.ShapeDtypeStruct((n, od), jnp.float32),
    )(a2f, a2l, aggf, aggl, crow)


NBUF = 2
SEG1 = 8


def _sc_segsum(m_flat, srcb, dstb, n_nodes, r, nbuf, seg):
    nbpc = srcb.shape[1]
    bps = nbpc // NS
    nseg = bps // seg
    assert bps % seg == 0 and seg % nbuf == 0 and seg >= 2 * nbuf
    npad = n_nodes + 8
    rows_zero = -(-npad // NS)
    rows_out = n_nodes // NS
    mesh = plsc.VectorSubcoreMesh(core_axis_name="c", subcore_axis_name="s")

    @functools.partial(
        pl.kernel,
        mesh=mesh,
        compiler_params=pltpu.CompilerParams(use_tc_tiling_on_sc=False),
        out_type=jax.ShapeDtypeStruct((2, n_nodes, r), jnp.float32),
        scratch_types=[
            pltpu.VMEM_SHARED((npad, r), jnp.float32),
            pltpu.VMEM((1 if nbuf == 1 else 2, seg, 128), jnp.int32),
            pltpu.VMEM((1 if nbuf == 1 else 2, seg, 128), jnp.int32),
        ]
        + [pltpu.VMEM((128, r), jnp.float32) for _ in range(nbuf)]
        + [pltpu.SemaphoreType.DMA for _ in range(2 * nbuf)]
        + [pltpu.SemaphoreType.DMA, pltpu.SemaphoreType.DMA],
    )
    def k(m_hbm, srcb_hbm, dstb_hbm, out_hbm, acc_sh, idxs, idxd, *rest):
        gbufs = rest[:nbuf]
        gsems = rest[nbuf:2 * nbuf]
        ssems = rest[2 * nbuf:3 * nbuf]
        isems = rest[3 * nbuf:]
        c = lax.axis_index("c")
        s = lax.axis_index("s")

        @pl.loop(0, 128)
        def _(i):
            @pl.loop(0, r, step=LANE)
            def _(j):
                gbufs[0][i, pl.ds(j, LANE)] = jnp.zeros((LANE,), jnp.float32)

        zbase = jnp.minimum(s * rows_zero, npad - rows_zero)
        off = 0
        left = rows_zero
        while left > 0:
            nn = min(128, left)
            pltpu.sync_copy(gbufs[0].at[pl.ds(0, nn)],
                            acc_sh.at[pl.ds(zbase + off, nn)])
            off += nn
            left -= nn
        plsc.subcore_barrier()

        b0 = s * bps

        def i_start(g, p):
            pltpu.async_copy(srcb_hbm.at[c].at[pl.ds(b0 + g * seg, seg)],
                             idxs.at[p], isems[0])
            pltpu.async_copy(dstb_hbm.at[c].at[pl.ds(b0 + g * seg, seg)],
                             idxd.at[p], isems[1])

        def i_wait(g, p):
            pltpu.make_async_copy(srcb_hbm.at[c].at[pl.ds(b0 + g * seg, seg)],
                                  idxs.at[p], isems[0]).wait()
            pltpu.make_async_copy(dstb_hbm.at[c].at[pl.ds(b0 + g * seg, seg)],
                                  idxd.at[p], isems[1]).wait()

        def g_start(p, jj, b):
            pltpu.async_copy(m_hbm.at[idxs.at[p].at[jj]], gbufs[b], gsems[b])

        def g_wait(p, jj, b):
            pltpu.make_async_copy(m_hbm.at[idxs.at[p].at[jj]], gbufs[b],
                                  gsems[b]).wait()

        def s_start(p, jj, b):
            pltpu.async_copy(gbufs[b], acc_sh.at[idxd.at[p].at[jj]], ssems[b],
                             add=True)

        def s_wait(p, jj, b):
            pltpu.make_async_copy(gbufs[b], acc_sh.at[idxd.at[p].at[jj]],
                                  ssems[b]).wait()

        assert nseg == 1 or nseg % 2 == 0
        i_start(0, 0)
        i_wait(0, 0)

        if nbuf == 1:
            @pl.loop(0, bps)
            def _(j):
                pltpu.sync_copy(m_hbm.at[idxs.at[0].at[j]], gbufs[0])
                pltpu.sync_copy(gbufs[0], acc_sh.at[idxd.at[0].at[j]], add=True)

            plsc.subcore_barrier()
            obase0 = s * rows_out
            pltpu.sync_copy(acc_sh.at[pl.ds(obase0, rows_out)],
                            out_hbm.at[c].at[pl.ds(obase0, rows_out)])
            return

        g_start(0, 0, 0)

        def maybe_when(cond, fn):
            if isinstance(cond, bool):
                if cond:
                    fn()
            else:
                pl.when(cond)(fn)

        def seg_body(g, p, pn, first):
            for jj in range(seg):
                b = jj % nbuf
                g_wait(p, jj, b)
                s_start(p, jj, b)
                bn = (jj + 1) % nbuf
                if first and jj + 1 < nbuf:
                    g_start(p, jj + 1, bn)
                elif jj + 1 < seg:
                    jprev = jj + 1 - nbuf
                    if first or jprev >= 0:
                        s_wait(p, jprev, bn)
                    else:
                        s_wait(pn, jprev + seg, bn)
                    g_start(p, jj + 1, bn)
                else:
                    def _refill():
                        i_wait(g + 1, pn)
                        s_wait(p, seg - nbuf, bn)
                        g_start(pn, 0, bn)
                    maybe_when(g < nseg - 1, _refill)
                if jj == nbuf - 1 and nseg > 1:
                    if first:
                        i_start(1, pn)
                    else:
                        maybe_when(g < nseg - 1,
                                   lambda: i_start(g + 1, pn))

        seg_body(0, 0, 1, True)
        if nseg > 2:
            @pl.loop(1, nseg - 1, step=2)
            def _(g):
                seg_body(g, 1, 0, False)
                seg_body(g + 1, 0, 1, False)
        if nseg > 1:
            seg_body(nseg - 1, (nseg - 1) % 2, nseg % 2, False)

        for b2 in range(nbuf):
            jj = seg - nbuf + b2
            s_wait((nseg - 1) % 2, jj, jj % nbuf)

        plsc.subcore_barrier()
        obase = s * rows_out
        pltpu.sync_copy(acc_sh.at[pl.ds(obase, rows_out)],
                        out_hbm.at[c].at[pl.ds(obase, rows_out)])

    return k(m_flat, srcb, dstb)


def _prep_idx(edge, n_nodes, split_edges, blk_mult):
    src, dst = edge[0], edge[1]
    e = src.shape[0]
    tot_workers = NS * (NC if split_edges else 1)
    per = -(-e // (tot_workers * 128 * blk_mult)) * 128 * blk_mult
    pad = per * tot_workers - e
    src_p = jnp.concatenate([src, jnp.zeros((pad,), jnp.int32)])
    dst_p = jnp.concatenate([dst, jnp.full((pad,), n_nodes, jnp.int32)])
    if split_edges:
        srcb = src_p.reshape(2, -1, 128)
        dstb = dst_p.reshape(2, -1, 128)
    else:
        srcb = jnp.stack([src_p, src_p + n_nodes]).reshape(2, -1, 128)
        dstb = jnp.stack([dst_p, dst_p]).reshape(2, -1, 128)
    return srcb, dstb


def kernel(x_user, x_item, edge_follows, edge_likes, edge_rev_likes,
           W0_follows, b0_follows, W0_likes, b0_likes, W0_rev_likes, b0_rev_likes,
           W1_follows, b1_follows, W1_likes, b1_likes, W1_rev_likes, b1_rev_likes,
           W_cls, b_cls):
    n = x_user.shape[0]
    h = W0_follows.shape[1]
    rb = 1000
    r1 = h // 2 + LANE

    mf = _mm_aug(x_user, W0_follows, rb)
    ml = _mm_aug(x_item, W0_likes, rb)
    mr = _mm_aug(x_user, W0_rev_likes, rb)

    sf1, df1 = _prep_idx(edge_follows, n, False, 1)
    sl1, dl1 = _prep_idx(edge_likes, n, False, 1)
    sr1, dr1 = _prep_idx(edge_rev_likes, n, False, 1)
    bps1 = sf1.shape[1] // NS
    aggf = _sc_segsum(mf, sf1, df1, n, r1, 1, bps1)
    aggl = _sc_segsum(ml, sl1, dl1, n, r1, 1, bps1)
    aggr = _sc_segsum(mr, sr1, dr1, n, r1, 1, bps1)

    wfp, wlp, crow = _fold(W1_follows, W1_likes, W_cls, b1_follows, b1_likes, b_cls)
    m2f = _relu2mm(aggf, aggl, b0_follows, b0_likes, wfp, rb)
    m2l = _relu1mm(aggr, b0_rev_likes, wlp, rb)

    sf2, df2 = _prep_idx(edge_follows, n, True, 4)
    sl2, dl2 = _prep_idx(edge_likes, n, True, 4)
    bps2 = sf2.shape[1] // NS
    a2f = _sc_segsum(m2f, sf2, df2, n, W_cls.shape[1], 4, bps2)
    a2l = _sc_segsum(m2l, sl2, dl2, n, W_cls.shape[1], 4, bps2)

    return _final(a2f, a2l, aggf, aggl, crow, rb)
